# Initial kernel scaffold; baseline (speedup 1.0000x reference)
import jax
import jax.numpy as jnp
from jax.experimental import pallas as pl


def kernel(input_ids, position_ids, word_embeddings, position_embeddings):
    raise NotImplementedError("write your pallas kernel here")


# speedup vs baseline: 2.2300x; 2.2300x over previous
import functools

import jax
import jax.numpy as jnp
from jax import lax
from jax.experimental import pallas as pl
from jax.experimental.pallas import tpu as pltpu
from jax.experimental.pallas import tpu_sc as plsc

EMBED = 64
CHUNK = 128


def _make_emb_kernel(n_rows: int):
    info = plsc.get_sparse_core_info()
    nc, ns = info.num_cores, info.num_subcores
    nw = nc * ns
    assert n_rows % (nw * CHUNK) == 0
    rpw = n_rows // nw
    n_chunks = rpw // CHUNK

    mesh = plsc.VectorSubcoreMesh(core_axis_name="c", subcore_axis_name="s")

    @functools.partial(
        pl.kernel,
        out_type=jax.ShapeDtypeStruct((n_rows, EMBED), jnp.float32),
        mesh=mesh,
        compiler_params=pltpu.CompilerParams(use_tc_tiling_on_sc=False),
        scratch_types=[
            pltpu.VMEM((rpw,), jnp.int32),
            pltpu.VMEM((rpw,), jnp.int32),
            pltpu.VMEM((CHUNK, EMBED), jnp.float32),
            pltpu.VMEM((CHUNK, EMBED), jnp.float32),
            pltpu.SemaphoreType.DMA,
            pltpu.SemaphoreType.DMA,
        ],
    )
    def emb_kernel(ids_hbm, pids_hbm, wtab_hbm, ptab_hbm, out_hbm,
                   idx_v, pidx_v, wbuf, pbuf, sem_w, sem_p):
        wid = lax.axis_index("s") * nc + lax.axis_index("c")
        base = wid * rpw
        pltpu.sync_copy(ids_hbm.at[pl.ds(base, rpw)], idx_v)
        pltpu.sync_copy(pids_hbm.at[pl.ds(base, rpw)], pidx_v)

        def chunk_body(i, carry):
            off = i * CHUNK
            cw = pltpu.async_copy(
                wtab_hbm.at[idx_v.at[pl.ds(off, CHUNK)]], wbuf, sem_w)
            cp = pltpu.async_copy(
                ptab_hbm.at[pidx_v.at[pl.ds(off, CHUNK)]], pbuf, sem_p)
            cw.wait()
            cp.wait()

            def add_row(r, c2):
                for c in range(0, EMBED, 16):
                    wbuf[r, pl.ds(c, 16)] = (
                        wbuf[r, pl.ds(c, 16)] + pbuf[r, pl.ds(c, 16)])
                return c2
            lax.fori_loop(0, CHUNK, add_row, 0)

            pltpu.sync_copy(wbuf, out_hbm.at[pl.ds(base + off, CHUNK)])
            return carry

        lax.fori_loop(0, n_chunks, chunk_body, 0)

    return emb_kernel


def kernel(input_ids, position_ids, word_embeddings, position_embeddings):
    b, l = input_ids.shape
    n = b * l
    ids = input_ids.reshape(n)
    pids = position_ids.reshape(n)
    emb = _make_emb_kernel(n)
    out = emb(ids, pids, word_embeddings, position_embeddings)
    return out.reshape(b, l, EMBED)


# speedup vs baseline: 2.3262x; 1.0431x over previous
import functools

import jax
import jax.numpy as jnp
from jax import lax
from jax.experimental import pallas as pl
from jax.experimental.pallas import tpu as pltpu
from jax.experimental.pallas import tpu_sc as plsc

EMBED = 64
CHUNK = 128
NBUF = 2


def _make_emb_kernel(n_rows: int):
    info = plsc.get_sparse_core_info()
    nc, ns = info.num_cores, info.num_subcores
    nw = nc * ns
    assert n_rows % (nw * CHUNK * NBUF) == 0
    rpw = n_rows // nw
    n_chunks = rpw // CHUNK
    n_groups = n_chunks // NBUF

    mesh = plsc.VectorSubcoreMesh(core_axis_name="c", subcore_axis_name="s")

    @functools.partial(
        pl.kernel,
        out_type=jax.ShapeDtypeStruct((n_rows, EMBED), jnp.float32),
        mesh=mesh,
        compiler_params=pltpu.CompilerParams(use_tc_tiling_on_sc=False),
        scratch_types=[
            pltpu.VMEM((rpw,), jnp.int32),
            pltpu.VMEM((rpw,), jnp.int32),
        ] + [pltpu.VMEM((CHUNK, EMBED), jnp.float32)] * (3 * NBUF)
          + [pltpu.SemaphoreType.DMA] * (3 * NBUF),
    )
    def emb_kernel(ids_hbm, pids_hbm, wtab_hbm, ptab_hbm, out_hbm,
                   idx_v, pidx_v, wbuf0, wbuf1, pbuf0, pbuf1, obuf0, obuf1,
                   sgw0, sgw1, sgp0, sgp1, ss0, ss1):
        wbufs, pbufs, obufs = (wbuf0, wbuf1), (pbuf0, pbuf1), (obuf0, obuf1)
        sgw, sgp, ss = (sgw0, sgw1), (sgp0, sgp1), (ss0, ss1)
        wid = lax.axis_index("s") * nc + lax.axis_index("c")
        base = wid * rpw
        pltpu.sync_copy(ids_hbm.at[pl.ds(base, rpw)], idx_v)
        pltpu.sync_copy(pids_hbm.at[pl.ds(base, rpw)], pidx_v)

        def gathers_start(j, b):
            off = j * CHUNK
            pltpu.async_copy(
                wtab_hbm.at[idx_v.at[pl.ds(off, CHUNK)]], wbufs[b], sgw[b])
            pltpu.async_copy(
                ptab_hbm.at[pidx_v.at[pl.ds(off, CHUNK)]], pbufs[b], sgp[b])

        def gathers_wait(b):
            pltpu.make_async_copy(
                wtab_hbm.at[idx_v.at[pl.ds(0, CHUNK)]], wbufs[b], sgw[b]).wait()
            pltpu.make_async_copy(
                ptab_hbm.at[pidx_v.at[pl.ds(0, CHUNK)]], pbufs[b], sgp[b]).wait()

        def add(b):
            def add_row(r, c2):
                for c in range(0, EMBED, 16):
                    obufs[b][r, pl.ds(c, 16)] = (
                        wbufs[b][r, pl.ds(c, 16)] + pbufs[b][r, pl.ds(c, 16)])
                return c2
            lax.fori_loop(0, CHUNK, add_row, 0)

        def scatter_start(j, b):
            pltpu.async_copy(
                obufs[b], out_hbm.at[pl.ds(base + j * CHUNK, CHUNK)], ss[b])

        def scatter_wait(b):
            pltpu.make_async_copy(
                obufs[b], out_hbm.at[pl.ds(base, CHUNK)], ss[b]).wait()

        for b in range(NBUF):
            gathers_start(b, b)
        for b in range(NBUF):
            gathers_wait(b)
            add(b)
            scatter_start(b, b)
            gathers_start(b + NBUF, b)

        def body(g, carry):
            for b in range(NBUF):
                j = g * NBUF + b
                gathers_wait(b)
                scatter_wait(b)
                add(b)
                scatter_start(j, b)
                gathers_start(j + NBUF, b)
            return carry
        lax.fori_loop(1, n_groups - 1, body, 0)

        for b in range(NBUF):
            gathers_wait(b)
            scatter_wait(b)
            add(b)
            scatter_start(n_chunks - NBUF + b, b)
        for b in range(NBUF):
            scatter_wait(b)

    return emb_kernel


def kernel(input_ids, position_ids, word_embeddings, position_embeddings):
    b, l = input_ids.shape
    n = b * l
    ids = input_ids.reshape(n)
    pids = position_ids.reshape(n)
    emb = _make_emb_kernel(n)
    out = emb(ids, pids, word_embeddings, position_embeddings)
    return out.reshape(b, l, EMBED)
